# Initial kernel scaffold; baseline (speedup 1.0000x reference)
import jax
import jax.numpy as jnp
from jax.experimental import pallas as pl


def kernel(x, w1, b1, g1, be1, w2, b2, g2, be2, w3, b3, g3, be3, w4, b4):
    raise NotImplementedError("write your pallas kernel here")


# speedup vs baseline: 4.8209x; 4.8209x over previous
import functools

import jax
import jax.numpy as jnp
from jax.experimental import pallas as pl
from jax.experimental.pallas import tpu as pltpu

_LEAKY = 0.01
_EPS = 1e-5
_TILE = 4096
_VMEM = 56 * 1024 * 1024


def _leaky(h):
    return jnp.where(h >= 0.0, h, jnp.float32(_LEAKY) * h)


def _colsum(a, ones_col):
    return jax.lax.dot_general(a, ones_col,
                               dimension_numbers=(((1,), (0,)), ((), ())),
                               preferred_element_type=jnp.float32)


def _fused_kernel(x_ref,
                  w1, b1c, g1c, be1c,
                  w2, b2c, g2c, be2c,
                  w3, b3c, g3c, be3c,
                  w4, b4,
                  out_ref,
                  a1t, a2t, a3t,
                  s1, q1, s2, q2, s3, q3,
                  *, nt, tile, rows_valid, rows_padded):
    i = pl.program_id(0)
    ones_col = jnp.ones((tile, 1), jnp.float32)
    padded = rows_valid != rows_padded

    def lane_mask(a, j):
        col = j * tile + jax.lax.broadcasted_iota(jnp.int32, a.shape, 1)
        return jnp.where(col < rows_valid, a, 0.0)

    @pl.when(i == 0)
    def _init():
        s1[...] = jnp.zeros_like(s1)
        q1[...] = jnp.zeros_like(q1)
        s2[...] = jnp.zeros_like(s2)
        q2[...] = jnp.zeros_like(q2)
        s3[...] = jnp.zeros_like(s3)
        q3[...] = jnp.zeros_like(q3)

    @pl.when(i < nt)
    def _stage1():
        h = jax.lax.dot_general(w1[...], x_ref[...],
                                dimension_numbers=(((0,), (1,)), ((), ())),
                                preferred_element_type=jnp.float32) + b1c[...]
        a = _leaky(h)
        a1t[i] = a
        am = lane_mask(a, i) if padded else a
        s1[...] += _colsum(am, ones_col)
        q1[...] += _colsum(am * am, ones_col)

    @pl.when(i == nt)
    def _rest():
        n = jnp.float32(rows_valid)

        def fold(s_ref, q_ref, g_ref, be_ref):
            mean = s_ref[...] / n
            var = jnp.maximum(q_ref[...] / n - mean * mean, 0.0)
            inv = jax.lax.rsqrt(var + _EPS)
            scale = g_ref[...] * inv
            shift = be_ref[...] - mean * scale
            return scale, shift

        def folded_layer(w_ref, b_ref, scale, shift):
            wf = w_ref[...] * scale
            bf = jax.lax.dot_general(
                w_ref[...], shift,
                dimension_numbers=(((0,), (0,)), ((), ())),
                preferred_element_type=jnp.float32) + b_ref[...]
            return wf, bf

        def mid_layer(src, dst, s_ref, q_ref, wf, bf):
            def body(j, carry):
                h = jax.lax.dot_general(
                    wf, src[j],
                    dimension_numbers=(((0,), (0,)), ((), ())),
                    preferred_element_type=jnp.float32) + bf
                a = _leaky(h)
                dst[j] = a
                am = lane_mask(a, j) if padded else a
                s_ref[...] += _colsum(am, ones_col)
                q_ref[...] += _colsum(am * am, ones_col)
                return carry
            jax.lax.fori_loop(0, nt, body, 0)

        sc1, sh1 = fold(s1, q1, g1c, be1c)
        w2f, b2f = folded_layer(w2, b2c, sc1, sh1)
        mid_layer(a1t, a2t, s2, q2, w2f, b2f)

        sc2, sh2 = fold(s2, q2, g2c, be2c)
        w3f, b3f = folded_layer(w3, b3c, sc2, sh2)
        mid_layer(a2t, a3t, s3, q3, w3f, b3f)

        sc3, sh3 = fold(s3, q3, g3c, be3c)
        w4f = w4[...] * sc3
        b4f = jnp.sum(sh3 * w4[...]) + b4[0, 0]

        def head_body(j, carry):
            z = jax.lax.dot_general(
                w4f, a3t[j],
                dimension_numbers=(((0,), (0,)), ((), ())),
                preferred_element_type=jnp.float32) + b4f
            out_ref[j] = 1.0 / (1.0 + jnp.exp(-z))
            return carry
        jax.lax.fori_loop(0, nt, head_body, 0)


def kernel(x, w1, b1, g1, be1, w2, b2, g2, be2, w3, b3, g3, be3, w4, b4):
    batch, k = x.shape
    f1, f2, f3 = w1.shape[1], w2.shape[1], w3.shape[1]
    tile = _TILE if batch >= _TILE else max(128, -(-batch // 128) * 128)
    nt = -(-batch // tile)
    rows_padded = nt * tile
    if rows_padded != batch:
        x = jnp.pad(x, ((0, rows_padded - batch), (0, 0)))

    full = lambda i: (0, 0)
    fused_fn = functools.partial(_fused_kernel, nt=nt, tile=tile,
                                 rows_valid=batch, rows_padded=rows_padded)
    out = pl.pallas_call(
        fused_fn,
        grid=(nt + 1,),
        in_specs=[
            pl.BlockSpec((tile, k), lambda i: (jnp.minimum(i, nt - 1), 0)),
            pl.BlockSpec((k, f1), full), pl.BlockSpec((f1, 1), full),
            pl.BlockSpec((f1, 1), full), pl.BlockSpec((f1, 1), full),
            pl.BlockSpec((f1, f2), full), pl.BlockSpec((f2, 1), full),
            pl.BlockSpec((f2, 1), full), pl.BlockSpec((f2, 1), full),
            pl.BlockSpec((f2, f3), full), pl.BlockSpec((f3, 1), full),
            pl.BlockSpec((f3, 1), full), pl.BlockSpec((f3, 1), full),
            pl.BlockSpec((f3, 1), full), pl.BlockSpec((1, 1), full),
        ],
        out_specs=pl.BlockSpec((nt, 1, tile), lambda i: (0, 0, 0)),
        out_shape=jax.ShapeDtypeStruct((nt, 1, tile), jnp.float32),
        scratch_shapes=[
            pltpu.VMEM((nt, f1, tile), jnp.float32),
            pltpu.VMEM((nt, f2, tile), jnp.float32),
            pltpu.VMEM((nt, f3, tile), jnp.float32),
            pltpu.VMEM((f1, 1), jnp.float32), pltpu.VMEM((f1, 1), jnp.float32),
            pltpu.VMEM((f2, 1), jnp.float32), pltpu.VMEM((f2, 1), jnp.float32),
            pltpu.VMEM((f3, 1), jnp.float32), pltpu.VMEM((f3, 1), jnp.float32),
        ],
        compiler_params=pltpu.CompilerParams(
            dimension_semantics=("arbitrary",),
            vmem_limit_bytes=_VMEM),
    )(x,
      w1, b1.reshape(f1, 1), g1.reshape(f1, 1), be1.reshape(f1, 1),
      w2, b2.reshape(f2, 1), g2.reshape(f2, 1), be2.reshape(f2, 1),
      w3, b3.reshape(f3, 1), g3.reshape(f3, 1), be3.reshape(f3, 1),
      w4, b4)
    return out.reshape(rows_padded, 1)[:batch]


# speedup vs baseline: 5.5993x; 1.1615x over previous
import functools

import jax
import jax.numpy as jnp
from jax.experimental import pallas as pl
from jax.experimental.pallas import tpu as pltpu

_LEAKY = 0.01
_EPS = 1e-5
_TILE = 4096
_VMEM = 56 * 1024 * 1024


def _leaky(h):
    return jnp.where(h >= 0.0, h, jnp.float32(_LEAKY) * h)


def _accum_stats(a, s_ref, q_ref):
    f, t = a.shape
    s = a[:, 0:128]
    q = s * s
    for c in range(128, t, 128):
        chunk = a[:, c:c + 128]
        s = s + chunk
        q = q + chunk * chunk
    s_ref[...] += s
    q_ref[...] += q


def _tdot(lhs, rhs):
    return jax.lax.dot_general(lhs, rhs,
                               dimension_numbers=(((0,), (0,)), ((), ())),
                               preferred_element_type=jnp.float32)


def _fused_kernel(x_ref,
                  w1, b1c, g1c, be1c,
                  w2, b2c, g2c, be2c,
                  w3, b3c, g3c, be3c,
                  w4, b4,
                  out_ref,
                  a1t, a2t, a3t,
                  s1, q1, s2, q2, s3, q3,
                  *, nt, tile, rows_valid, rows_padded):
    i = pl.program_id(0)
    padded = rows_valid != rows_padded

    def lane_mask(a, j):
        col = j * tile + jax.lax.broadcasted_iota(jnp.int32, a.shape, 1)
        return jnp.where(col < rows_valid, a, 0.0)

    @pl.when(i == 0)
    def _init():
        s1[...] = jnp.zeros_like(s1)
        q1[...] = jnp.zeros_like(q1)
        s2[...] = jnp.zeros_like(s2)
        q2[...] = jnp.zeros_like(q2)
        s3[...] = jnp.zeros_like(s3)
        q3[...] = jnp.zeros_like(q3)

    @pl.when(i < nt)
    def _stage1():
        h = jax.lax.dot_general(w1[...].astype(jnp.bfloat16),
                                x_ref[...].astype(jnp.bfloat16),
                                dimension_numbers=(((0,), (1,)), ((), ())),
                                preferred_element_type=jnp.float32) + b1c[...]
        a = _leaky(h)
        a1t[i] = a.astype(jnp.bfloat16)
        am = lane_mask(a, i) if padded else a
        _accum_stats(am, s1, q1)

    @pl.when(i == nt)
    def _rest():
        n = jnp.float32(rows_valid)

        def fold(s_ref, q_ref, g_ref, be_ref):
            s = jnp.sum(s_ref[...], axis=1, keepdims=True)
            q = jnp.sum(q_ref[...], axis=1, keepdims=True)
            mean = s / n
            var = jnp.maximum(q / n - mean * mean, 0.0)
            inv = jax.lax.rsqrt(var + _EPS)
            scale = g_ref[...] * inv
            shift = be_ref[...] - mean * scale
            return scale, shift

        def folded_layer(w_ref, b_ref, scale, shift):
            wf = (w_ref[...] * scale).astype(jnp.bfloat16)
            bf = _tdot(w_ref[...], shift) + b_ref[...]
            return wf, bf

        def mid_layer(src, dst, s_ref, q_ref, wf, bf):
            def body(j, carry):
                h = _tdot(wf, src[j]) + bf
                a = _leaky(h)
                dst[j] = a.astype(jnp.bfloat16)
                am = lane_mask(a, j) if padded else a
                _accum_stats(am, s_ref, q_ref)
                return carry
            jax.lax.fori_loop(0, nt, body, 0)

        sc1, sh1 = fold(s1, q1, g1c, be1c)
        w2f, b2f = folded_layer(w2, b2c, sc1, sh1)
        mid_layer(a1t, a2t, s2, q2, w2f, b2f)

        sc2, sh2 = fold(s2, q2, g2c, be2c)
        w3f, b3f = folded_layer(w3, b3c, sc2, sh2)
        mid_layer(a2t, a3t, s3, q3, w3f, b3f)

        sc3, sh3 = fold(s3, q3, g3c, be3c)
        w4f = (w4[...] * sc3).astype(jnp.bfloat16)
        b4f = jnp.sum(sh3 * w4[...]) + b4[0, 0]

        def head_body(j, carry):
            z = _tdot(w4f, a3t[j]) + b4f
            out_ref[j] = 1.0 / (1.0 + jnp.exp(-z))
            return carry
        jax.lax.fori_loop(0, nt, head_body, 0)


def kernel(x, w1, b1, g1, be1, w2, b2, g2, be2, w3, b3, g3, be3, w4, b4):
    batch, k = x.shape
    f1, f2, f3 = w1.shape[1], w2.shape[1], w3.shape[1]
    tile = _TILE if batch >= _TILE else max(128, -(-batch // 128) * 128)
    nt = -(-batch // tile)
    rows_padded = nt * tile
    if rows_padded != batch:
        x = jnp.pad(x, ((0, rows_padded - batch), (0, 0)))

    full = lambda i: (0, 0)
    fused_fn = functools.partial(_fused_kernel, nt=nt, tile=tile,
                                 rows_valid=batch, rows_padded=rows_padded)
    out = pl.pallas_call(
        fused_fn,
        grid=(nt + 1,),
        in_specs=[
            pl.BlockSpec((tile, k), lambda i: (jnp.minimum(i, nt - 1), 0)),
            pl.BlockSpec((k, f1), full), pl.BlockSpec((f1, 1), full),
            pl.BlockSpec((f1, 1), full), pl.BlockSpec((f1, 1), full),
            pl.BlockSpec((f1, f2), full), pl.BlockSpec((f2, 1), full),
            pl.BlockSpec((f2, 1), full), pl.BlockSpec((f2, 1), full),
            pl.BlockSpec((f2, f3), full), pl.BlockSpec((f3, 1), full),
            pl.BlockSpec((f3, 1), full), pl.BlockSpec((f3, 1), full),
            pl.BlockSpec((f3, 1), full), pl.BlockSpec((1, 1), full),
        ],
        out_specs=pl.BlockSpec((nt, 1, tile), lambda i: (0, 0, 0)),
        out_shape=jax.ShapeDtypeStruct((nt, 1, tile), jnp.float32),
        scratch_shapes=[
            pltpu.VMEM((nt, f1, tile), jnp.bfloat16),
            pltpu.VMEM((nt, f2, tile), jnp.bfloat16),
            pltpu.VMEM((nt, f3, tile), jnp.bfloat16),
            pltpu.VMEM((f1, 128), jnp.float32), pltpu.VMEM((f1, 128), jnp.float32),
            pltpu.VMEM((f2, 128), jnp.float32), pltpu.VMEM((f2, 128), jnp.float32),
            pltpu.VMEM((f3, 128), jnp.float32), pltpu.VMEM((f3, 128), jnp.float32),
        ],
        compiler_params=pltpu.CompilerParams(
            dimension_semantics=("arbitrary",),
            vmem_limit_bytes=_VMEM),
    )(x,
      w1, b1.reshape(f1, 1), g1.reshape(f1, 1), be1.reshape(f1, 1),
      w2, b2.reshape(f2, 1), g2.reshape(f2, 1), be2.reshape(f2, 1),
      w3, b3.reshape(f3, 1), g3.reshape(f3, 1), be3.reshape(f3, 1),
      w4, b4)
    return out.reshape(rows_padded, 1)[:batch]


# speedup vs baseline: 6.1631x; 1.1007x over previous
import functools

import jax
import jax.numpy as jnp
from jax.experimental import pallas as pl
from jax.experimental.pallas import tpu as pltpu

_LEAKY = 0.01
_EPS = 1e-5
_TILE = 4096
_VMEM = 56 * 1024 * 1024


def _leaky(h):
    return jnp.where(h >= 0.0, h, jnp.float32(_LEAKY) * h)


def _tree_sum(vals):
    while len(vals) > 1:
        pairs = [vals[i] + vals[i + 1] for i in range(0, len(vals) - 1, 2)]
        if len(vals) % 2:
            pairs.append(vals[-1])
        vals = pairs
    return vals[0]


def _accum_stats(a, s_ref, q_ref):
    f, t = a.shape
    chunks = [a[:, c:c + 128] for c in range(0, t, 128)]
    s_ref[...] += _tree_sum(chunks)
    q_ref[...] += _tree_sum([c * c for c in chunks])


def _tdot(lhs, rhs):
    return jax.lax.dot_general(lhs, rhs,
                               dimension_numbers=(((0,), (0,)), ((), ())),
                               preferred_element_type=jnp.float32)


def _col(row):
    eye = (jax.lax.broadcasted_iota(jnp.int32, (128, 128), 0) ==
           jax.lax.broadcasted_iota(jnp.int32, (128, 128), 1)
           ).astype(jnp.float32)
    return jax.lax.dot_general(eye, row,
                               dimension_numbers=(((1,), (1,)), ((), ())),
                               preferred_element_type=jnp.float32)


_B1, _G1, _BE1, _B2, _G2, _BE2, _B3, _G3, _BE3, _W4, _B4 = range(11)


def _fused_kernel(x_ref, w1, w2, w3, vecs,
                  out_ref,
                  a1t, a2t, a3t,
                  s1, q1, s2, q2, s3, q3,
                  *, nt, tile, f1, f2, f3, rows_valid, rows_padded):
    i = pl.program_id(0)
    padded = rows_valid != rows_padded

    def vcol(r, f):
        return _col(vecs[r:r + 1, :])[:f]

    def lane_mask(a, j):
        col = j * tile + jax.lax.broadcasted_iota(jnp.int32, a.shape, 1)
        return jnp.where(col < rows_valid, a, 0.0)

    @pl.when(i == 0)
    def _init():
        s1[...] = jnp.zeros_like(s1)
        q1[...] = jnp.zeros_like(q1)
        s2[...] = jnp.zeros_like(s2)
        q2[...] = jnp.zeros_like(q2)
        s3[...] = jnp.zeros_like(s3)
        q3[...] = jnp.zeros_like(q3)

    @pl.when(i < nt)
    def _stage1():
        h = jax.lax.dot_general(w1[...].astype(jnp.bfloat16),
                                x_ref[...].astype(jnp.bfloat16),
                                dimension_numbers=(((0,), (1,)), ((), ())),
                                preferred_element_type=jnp.float32)
        h = h + vcol(_B1, f1)
        a = _leaky(h)
        a1t[i] = a.astype(jnp.bfloat16)
        am = lane_mask(a, i) if padded else a
        _accum_stats(am, s1, q1)

    @pl.when(i == nt)
    def _rest():
        n = jnp.float32(rows_valid)

        def fold(s_ref, q_ref, g_row, be_row, f):
            s = jnp.sum(s_ref[...], axis=1, keepdims=True)
            q = jnp.sum(q_ref[...], axis=1, keepdims=True)
            mean = s / n
            var = jnp.maximum(q / n - mean * mean, 0.0)
            inv = jax.lax.rsqrt(var + _EPS)
            scale = vcol(g_row, f) * inv
            shift = vcol(be_row, f) - mean * scale
            return scale, shift

        def folded_layer(w_ref, b_row, f_out, scale, shift):
            wf = (w_ref[...] * scale).astype(jnp.bfloat16)
            bf = _tdot(w_ref[...], shift) + vcol(b_row, f_out)
            return wf, bf

        def mid_layer(src, dst, s_ref, q_ref, wf, bf):
            def body(j, carry):
                h = _tdot(wf, src[j]) + bf
                a = _leaky(h)
                dst[j] = a.astype(jnp.bfloat16)
                am = lane_mask(a, j) if padded else a
                _accum_stats(am, s_ref, q_ref)
                return carry
            jax.lax.fori_loop(0, nt, body, 0)

        sc1, sh1 = fold(s1, q1, _G1, _BE1, f1)
        w2f, b2f = folded_layer(w2, _B2, f2, sc1, sh1)
        mid_layer(a1t, a2t, s2, q2, w2f, b2f)

        sc2, sh2 = fold(s2, q2, _G2, _BE2, f2)
        w3f, b3f = folded_layer(w3, _B3, f3, sc2, sh2)
        mid_layer(a2t, a3t, s3, q3, w3f, b3f)

        sc3, sh3 = fold(s3, q3, _G3, _BE3, f3)
        w4c = vcol(_W4, f3)
        w4f = (w4c * sc3).astype(jnp.bfloat16)
        b4f = (jnp.sum(sh3 * w4c, axis=0, keepdims=True)
               + vecs[_B4:_B4 + 1, 0:1])

        def head_body(j, carry):
            z = _tdot(w4f, a3t[j]) + b4f
            out_ref[j] = 1.0 / (1.0 + jnp.exp(-z))
            return carry
        jax.lax.fori_loop(0, nt, head_body, 0)


def kernel(x, w1, b1, g1, be1, w2, b2, g2, be2, w3, b3, g3, be3, w4, b4):
    batch, k = x.shape
    f1, f2, f3 = w1.shape[1], w2.shape[1], w3.shape[1]
    tile = _TILE if batch >= _TILE else max(128, -(-batch // 128) * 128)
    nt = -(-batch // tile)
    rows_padded = nt * tile
    if rows_padded != batch:
        x = jnp.pad(x, ((0, rows_padded - batch), (0, 0)))

    def row(v):
        r = v.reshape(1, -1)
        return jnp.pad(r, ((0, 0), (0, 128 - r.shape[1])))

    vecs = jnp.concatenate(
        [row(b1), row(g1), row(be1), row(b2), row(g2), row(be2),
         row(b3), row(g3), row(be3), row(w4), row(b4)], axis=0)

    full = lambda i: (0, 0)
    fused_fn = functools.partial(_fused_kernel, nt=nt, tile=tile,
                                 f1=f1, f2=f2, f3=f3,
                                 rows_valid=batch, rows_padded=rows_padded)
    out = pl.pallas_call(
        fused_fn,
        grid=(nt + 1,),
        in_specs=[
            pl.BlockSpec((tile, k), lambda i: (jnp.minimum(i, nt - 1), 0)),
            pl.BlockSpec((k, f1), full),
            pl.BlockSpec((f1, f2), full),
            pl.BlockSpec((f2, f3), full),
            pl.BlockSpec((11, 128), full),
        ],
        out_specs=pl.BlockSpec((nt, 1, tile), lambda i: (0, 0, 0)),
        out_shape=jax.ShapeDtypeStruct((nt, 1, tile), jnp.float32),
        scratch_shapes=[
            pltpu.VMEM((nt, f1, tile), jnp.bfloat16),
            pltpu.VMEM((nt, f2, tile), jnp.bfloat16),
            pltpu.VMEM((nt, f3, tile), jnp.bfloat16),
            pltpu.VMEM((f1, 128), jnp.float32), pltpu.VMEM((f1, 128), jnp.float32),
            pltpu.VMEM((f2, 128), jnp.float32), pltpu.VMEM((f2, 128), jnp.float32),
            pltpu.VMEM((f3, 128), jnp.float32), pltpu.VMEM((f3, 128), jnp.float32),
        ],
        compiler_params=pltpu.CompilerParams(
            dimension_semantics=("arbitrary",),
            vmem_limit_bytes=_VMEM),
    )(x, w1, w2, w3, vecs)
    return out.reshape(rows_padded, 1)[:batch]


# speedup vs baseline: 8.3459x; 1.3542x over previous
import functools

import jax
import jax.numpy as jnp
from jax.experimental import pallas as pl
from jax.experimental.pallas import tpu as pltpu

_LEAKY = 0.01
_EPS = 1e-5
_TILE = 4096
_VMEM = 56 * 1024 * 1024


def _leaky(h):
    return jnp.where(h >= 0.0, h, jnp.float32(_LEAKY) * h)


def _tree_sum(vals):
    while len(vals) > 1:
        pairs = [vals[i] + vals[i + 1] for i in range(0, len(vals) - 1, 2)]
        if len(vals) % 2:
            pairs.append(vals[-1])
        vals = pairs
    return vals[0]


def _stats(a):
    f, t = a.shape
    chunks = [a[:, c:c + 128] for c in range(0, t, 128)]
    return _tree_sum(chunks), _tree_sum([c * c for c in chunks])


def _tdot(lhs, rhs):
    return jax.lax.dot_general(lhs, rhs,
                               dimension_numbers=(((0,), (0,)), ((), ())),
                               preferred_element_type=jnp.float32)


def _col(row_ref):
    f = row_ref.shape[1]
    eye = (jax.lax.broadcasted_iota(jnp.int32, (f, f), 0) ==
           jax.lax.broadcasted_iota(jnp.int32, (f, f), 1)).astype(jnp.float32)
    return jax.lax.dot_general(eye, row_ref[...],
                               dimension_numbers=(((1,), (1,)), ((), ())),
                               preferred_element_type=jnp.float32)


def _fused_kernel(x_ref, w1, b1r, g1r, be1r, w2, b2r, g2r, be2r,
                  w3, b3r, g3r, be3r, w4r, b4,
                  out_ref,
                  a1t, a2t, a3t, s1, q1,
                  *, nt, tile, rows_valid, rows_padded):
    i = pl.program_id(0)
    padded = rows_valid != rows_padded
    n = jnp.float32(rows_valid)

    def lane_mask(a, j):
        col = j * tile + jax.lax.broadcasted_iota(jnp.int32, a.shape, 1)
        return jnp.where(col < rows_valid, a, 0.0)

    @pl.when(i == 0)
    def _init():
        s1[...] = jnp.zeros_like(s1)
        q1[...] = jnp.zeros_like(q1)

    @pl.when(i < nt)
    def _stage1():
        h = jax.lax.dot_general(w1[...].astype(jnp.bfloat16),
                                x_ref[...].astype(jnp.bfloat16),
                                dimension_numbers=(((0,), (1,)), ((), ())),
                                preferred_element_type=jnp.float32)
        a = _leaky(h + _col(b1r))
        a1t[i] = a.astype(jnp.bfloat16)
        am = lane_mask(a, i) if padded else a
        ds, dq = _stats(am)
        s1[...] += ds
        q1[...] += dq

    @pl.when(i == nt)
    def _rest():
        def fold(s, q, g_row, be_row):
            mean = s / n
            var = jnp.maximum(q / n - mean * mean, 0.0)
            inv = jax.lax.rsqrt(var + _EPS)
            scale = _col(g_row) * inv
            shift = _col(be_row) - mean * scale
            return scale, shift

        def folded_layer(w_ref, b_row, scale, shift):
            wf = (w_ref[...] * scale).astype(jnp.bfloat16)
            bf = _tdot(w_ref[...], shift) + _col(b_row)
            return wf, bf

        def mid_layer(src, dst, wf, bf):
            s = q = None
            for j in range(nt):
                h = _tdot(wf, src[j]) + bf
                a = _leaky(h)
                dst[j] = a.astype(jnp.bfloat16)
                am = lane_mask(a, j) if padded else a
                ds, dq = _stats(am)
                s = ds if s is None else s + ds
                q = dq if q is None else q + dq
            return (jnp.sum(s, axis=1, keepdims=True),
                    jnp.sum(q, axis=1, keepdims=True))

        s1c = jnp.sum(s1[...], axis=1, keepdims=True)
        q1c = jnp.sum(q1[...], axis=1, keepdims=True)
        sc1, sh1 = fold(s1c, q1c, g1r, be1r)
        w2f, b2f = folded_layer(w2, b2r, sc1, sh1)
        s2c, q2c = mid_layer(a1t, a2t, w2f, b2f)

        sc2, sh2 = fold(s2c, q2c, g2r, be2r)
        w3f, b3f = folded_layer(w3, b3r, sc2, sh2)
        s3c, q3c = mid_layer(a2t, a3t, w3f, b3f)

        sc3, sh3 = fold(s3c, q3c, g3r, be3r)
        w4c = _col(w4r)
        w4f = (w4c * sc3).astype(jnp.bfloat16)
        b4f = jnp.sum(sh3 * w4c, axis=0, keepdims=True) + b4[...]

        for j in range(nt):
            z = _tdot(w4f, a3t[j]) + b4f
            out_ref[j] = 1.0 / (1.0 + jnp.exp(-z))


def kernel(x, w1, b1, g1, be1, w2, b2, g2, be2, w3, b3, g3, be3, w4, b4):
    batch, k = x.shape
    f1, f2, f3 = w1.shape[1], w2.shape[1], w3.shape[1]
    tile = _TILE if batch >= _TILE else max(128, -(-batch // 128) * 128)
    nt = -(-batch // tile)
    rows_padded = nt * tile
    if rows_padded != batch:
        x = jnp.pad(x, ((0, rows_padded - batch), (0, 0)))

    full = lambda i: (0, 0)
    rspec = lambda f: pl.BlockSpec((1, f), full)
    fused_fn = functools.partial(_fused_kernel, nt=nt, tile=tile,
                                 rows_valid=batch, rows_padded=rows_padded)
    out = pl.pallas_call(
        fused_fn,
        grid=(nt + 1,),
        in_specs=[
            pl.BlockSpec((tile, k), lambda i: (jnp.minimum(i, nt - 1), 0)),
            pl.BlockSpec((k, f1), full), rspec(f1), rspec(f1), rspec(f1),
            pl.BlockSpec((f1, f2), full), rspec(f2), rspec(f2), rspec(f2),
            pl.BlockSpec((f2, f3), full), rspec(f3), rspec(f3), rspec(f3),
            rspec(f3), pl.BlockSpec((1, 1), full),
        ],
        out_specs=pl.BlockSpec((nt, 1, tile), lambda i: (0, 0, 0)),
        out_shape=jax.ShapeDtypeStruct((nt, 1, tile), jnp.float32),
        scratch_shapes=[
            pltpu.VMEM((nt, f1, tile), jnp.bfloat16),
            pltpu.VMEM((nt, f2, tile), jnp.bfloat16),
            pltpu.VMEM((nt, f3, tile), jnp.bfloat16),
            pltpu.VMEM((f1, 128), jnp.float32),
            pltpu.VMEM((f1, 128), jnp.float32),
        ],
        compiler_params=pltpu.CompilerParams(
            dimension_semantics=("arbitrary",),
            vmem_limit_bytes=_VMEM),
    )(x, w1, b1, g1.reshape(1, f1), be1.reshape(1, f1),
      w2, b2, g2.reshape(1, f2), be2.reshape(1, f2),
      w3, b3, g3.reshape(1, f3), be3.reshape(1, f3),
      w4.reshape(1, f3), b4)
    return out.reshape(rows_padded, 1)[:batch]
